# baseline (speedup 1.0000x reference)
import functools

import jax
import jax.numpy as jnp
from jax import lax
from jax.experimental import pallas as pl
from jax.experimental.pallas import tpu as pltpu
from jax.experimental.pallas import tpu_sc as plsc

D = 128
NF = 4
LANES = 16
NC, NS = 2, 16
NW = NC * NS


def _prep_body(table_ref, wf_ref, bfc_ref, wc_ref, tbl_out_ref, wfc_out_ref):
    wc1 = wc_ref[:D, :]
    wc2 = wc_ref[D:, :]
    bias = (
        jnp.dot(bfc_ref[0:1, :], wc2, preferred_element_type=jnp.float32)
        + bfc_ref[1:2, :]
    )
    tbl_out_ref[...] = (
        jnp.dot(table_ref[...], wc1, preferred_element_type=jnp.float32) + bias
    )
    wfc_out_ref[...] = jnp.dot(wf_ref[...], wc2, preferred_element_type=jnp.float32)


def _prep(table, Wf, bfc, Wc):
    num_atoms = table.shape[0]
    return pl.pallas_call(
        _prep_body,
        out_shape=[
            jax.ShapeDtypeStruct((num_atoms, D), jnp.float32),
            jax.ShapeDtypeStruct((NF, D), jnp.float32),
        ],
    )(table, Wf, bfc, Wc)


@functools.partial(jax.jit, static_argnames=("num_atoms", "batch", "seq"))
def _sc_lookup(ids, feat, tbl, wfc, *, num_atoms, batch, seq):
    n_rows = batch * seq
    rows_per_w = n_rows // NW
    batches_per_w = batch // NW
    chunk = seq
    n_full_groups = chunk // LANES
    tail = chunk - n_full_groups * LANES
    mesh = plsc.VectorSubcoreMesh(
        core_axis_name="c", subcore_axis_name="s", num_cores=NC, num_subcores=NS
    )

    @functools.partial(
        pl.kernel,
        out_type=jax.ShapeDtypeStruct((batch, seq, D), jnp.float32),
        mesh=mesh,
        compiler_params=pltpu.CompilerParams(needs_layout_passes=False),
        scratch_types=[
            pltpu.VMEM((num_atoms * D,), jnp.float32),
            pltpu.VMEM((NF, D), jnp.float32),
            pltpu.VMEM((chunk + LANES,), jnp.int32),
            pltpu.VMEM((chunk * NF + LANES,), jnp.float32),
            pltpu.VMEM((chunk, D), jnp.float32),
        ],
    )
    def body(ids_hbm, feat_hbm, tbl_hbm, wfc_hbm, out_hbm, tbl_v, wfc_v, ids_v, feat_v, out_v):
        wid = lax.axis_index("s") * NC + lax.axis_index("c")
        base0 = wid * rows_per_w
        b0 = wid * batches_per_w
        pltpu.sync_copy(tbl_hbm, tbl_v)
        pltpu.sync_copy(wfc_hbm, wfc_v)
        cols = [lax.iota(jnp.int32, LANES) + LANES * j for j in range(D // LANES)]
        w = [
            [wfc_v[k, pl.ds(LANES * j, LANES)] for j in range(D // LANES)]
            for k in range(NF)
        ]

        def do_group(r0):
            idbase = ids_v[pl.ds(r0, LANES)] * D
            fq = [feat_v[pl.ds(r0 * NF + q * LANES, LANES)] for q in range(4)]
            for rr in range(LANES):
                f = fq[rr // 4]
                f0 = f[(rr % 4) * NF + 0]
                f1 = f[(rr % 4) * NF + 1]
                f2 = f[(rr % 4) * NF + 2]
                f3 = f[(rr % 4) * NF + 3]
                for j in range(D // LANES):
                    g16 = plsc.load_gather(tbl_v, [idbase[rr] + cols[j]])
                    acc = (g16 + (f0 * w[0][j] + f1 * w[1][j])) + (
                        f2 * w[2][j] + f3 * w[3][j]
                    )
                    out_v[r0 + rr, pl.ds(LANES * j, LANES)] = acc

        def chunk_body(it, carry):
            base = base0 + it * chunk
            pltpu.sync_copy(ids_hbm.at[pl.ds(base, chunk)], ids_v.at[pl.ds(0, chunk)])
            pltpu.sync_copy(
                feat_hbm.at[pl.ds(base * NF, chunk * NF)],
                feat_v.at[pl.ds(0, chunk * NF)],
            )

            @plsc.parallel_loop(0, n_full_groups, unroll=2)
            def group_body(g):
                do_group(g * LANES)

            if tail:
                do_group(chunk - LANES)

            pltpu.sync_copy(out_v, out_hbm.at[b0 + it])
            return carry

        lax.fori_loop(0, batches_per_w, chunk_body, 0)

    return body(ids, feat, tbl, wfc)


def kernel(atom_ids, atom_features, table, Wf, bf, Wc, bc):
    B, L = atom_ids.shape
    n_rows = B * L
    ids = atom_ids.reshape(n_rows).astype(jnp.int32)
    feat = atom_features.reshape(n_rows * NF)
    bfc = jnp.stack([bf, bc])
    tbl, wfc = _prep(table, Wf, bfc, Wc)
    return _sc_lookup(
        ids, feat, tbl.reshape(-1), wfc,
        num_atoms=table.shape[0], batch=B, seq=L,
    )


# speedup vs baseline: 1.0522x; 1.0522x over previous
import functools

import jax
import jax.numpy as jnp
from jax import lax
from jax.experimental import pallas as pl
from jax.experimental.pallas import tpu as pltpu
from jax.experimental.pallas import tpu_sc as plsc

D = 128
NF = 4
LANES = 16
NC, NS = 2, 16
NW = NC * NS


def _prep_body(table_ref, wf_ref, bfc_ref, wc_ref, tbl_out_ref, wfc_out_ref):
    wc1 = wc_ref[:D, :]
    wc2 = wc_ref[D:, :]
    bias = (
        jnp.dot(bfc_ref[0:1, :], wc2, preferred_element_type=jnp.float32)
        + bfc_ref[1:2, :]
    )
    tbl_out_ref[...] = (
        jnp.dot(table_ref[...], wc1, preferred_element_type=jnp.float32) + bias
    )
    wfc_out_ref[...] = jnp.dot(wf_ref[...], wc2, preferred_element_type=jnp.float32)


def _prep(table, Wf, bfc, Wc):
    num_atoms = table.shape[0]
    return pl.pallas_call(
        _prep_body,
        out_shape=[
            jax.ShapeDtypeStruct((num_atoms, D), jnp.float32),
            jax.ShapeDtypeStruct((NF, D), jnp.float32),
        ],
    )(table, Wf, bfc, Wc)


@functools.partial(jax.jit, static_argnames=("num_atoms", "n_rows", "chunk", "sub"))
def _sc_lookup(ids, feat, tbl, wfc, *, num_atoms, n_rows, chunk, sub):
    rows_per_w = n_rows // NW
    n_chunks = rows_per_w // chunk
    n_sub = chunk // sub
    id_rows = chunk // D
    f_rows = chunk * NF // D
    mesh = plsc.VectorSubcoreMesh(
        core_axis_name="c", subcore_axis_name="s", num_cores=NC, num_subcores=NS
    )

    @functools.partial(
        pl.kernel,
        out_type=jax.ShapeDtypeStruct((n_rows, D), jnp.float32),
        mesh=mesh,
        compiler_params=pltpu.CompilerParams(needs_layout_passes=False),
        scratch_types=[
            pltpu.VMEM((num_atoms * D,), jnp.float32),
            pltpu.VMEM((NF, D), jnp.float32),
            pltpu.VMEM((id_rows, D), jnp.int32),
            pltpu.VMEM((f_rows, D), jnp.float32),
            pltpu.VMEM((sub, D), jnp.float32),
        ],
    )
    def body(ids_hbm, feat_hbm, tbl_hbm, wfc_hbm, out_hbm, tbl_v, wfc_v, ids_v, feat_v, out_v):
        wid = lax.axis_index("s") * NC + lax.axis_index("c")
        base0 = wid * rows_per_w
        pltpu.sync_copy(tbl_hbm, tbl_v)
        pltpu.sync_copy(wfc_hbm, wfc_v)
        cols = [lax.iota(jnp.int32, LANES) + LANES * j for j in range(D // LANES)]
        w = [
            [wfc_v[k, pl.ds(LANES * j, LANES)] for j in range(D // LANES)]
            for k in range(NF)
        ]

        def do_group(g, out_r0):
            r0 = g * LANES
            idbase = ids_v[r0 // D, pl.ds(r0 % D, LANES)] * D
            fq = [
                feat_v[
                    (r0 * NF + q * LANES) // D,
                    pl.ds((r0 * NF + q * LANES) % D, LANES),
                ]
                for q in range(4)
            ]
            for rr in range(LANES):
                f = fq[rr // 4]
                f0 = f[(rr % 4) * NF + 0]
                f1 = f[(rr % 4) * NF + 1]
                f2 = f[(rr % 4) * NF + 2]
                f3 = f[(rr % 4) * NF + 3]
                for j in range(D // LANES):
                    g16 = plsc.load_gather(tbl_v, [idbase[rr] + cols[j]])
                    acc = (g16 + (f0 * w[0][j] + f1 * w[1][j])) + (
                        f2 * w[2][j] + f3 * w[3][j]
                    )
                    out_v[out_r0 + rr, pl.ds(LANES * j, LANES)] = acc

        def chunk_body(it, carry):
            base = base0 + it * chunk
            pltpu.sync_copy(
                ids_hbm.at[pl.ds(pl.multiple_of(base // D, 8), id_rows)], ids_v
            )
            pltpu.sync_copy(
                feat_hbm.at[pl.ds(pl.multiple_of(base * NF // D, 8), f_rows)], feat_v
            )

            for c in range(n_sub):
                @plsc.parallel_loop(0, sub // LANES, unroll=2)
                def group_body(gg):
                    do_group(c * (sub // LANES) + gg, gg * LANES)

                pltpu.sync_copy(out_v, out_hbm.at[pl.ds(base + c * sub, sub)])
            return carry

        lax.fori_loop(0, n_chunks, chunk_body, 0)

    return body(ids, feat, tbl, wfc)


def kernel(atom_ids, atom_features, table, Wf, bf, Wc, bc):
    B, L = atom_ids.shape
    n_rows = B * L
    ids = atom_ids.reshape(n_rows // D, D).astype(jnp.int32)
    feat = atom_features.reshape(n_rows * NF // D, D)
    bfc = jnp.stack([bf, bc])
    tbl, wfc = _prep(table, Wf, bfc, Wc)
    out = _sc_lookup(
        ids, feat, tbl.reshape(-1), wfc,
        num_atoms=table.shape[0], n_rows=n_rows, chunk=1024, sub=256,
    )
    return out.reshape(B, L, D)


# speedup vs baseline: 1.2705x; 1.2075x over previous
import functools

import jax
import jax.numpy as jnp
from jax import lax
from jax.experimental import pallas as pl
from jax.experimental.pallas import tpu as pltpu
from jax.experimental.pallas import tpu_sc as plsc

D = 128
NF = 4
LANES = 16
NC, NS = 2, 16
NW = NC * NS


def _prep_body(table_ref, wf_ref, bfc_ref, wc_ref, tbl_out_ref, wfc_out_ref):
    wc1 = wc_ref[:D, :]
    wc2 = wc_ref[D:, :]
    bias = (
        jnp.dot(bfc_ref[0:1, :], wc2, preferred_element_type=jnp.float32)
        + bfc_ref[1:2, :]
    )
    tbl_out_ref[...] = (
        jnp.dot(table_ref[...], wc1, preferred_element_type=jnp.float32) + bias
    )
    wfc_out_ref[...] = jnp.dot(wf_ref[...], wc2, preferred_element_type=jnp.float32)


def _prep(table, Wf, bfc, Wc):
    num_atoms = table.shape[0]
    return pl.pallas_call(
        _prep_body,
        out_shape=[
            jax.ShapeDtypeStruct((num_atoms, D), jnp.float32),
            jax.ShapeDtypeStruct((NF, D), jnp.float32),
        ],
    )(table, Wf, bfc, Wc)


@functools.partial(jax.jit, static_argnames=("num_atoms", "batch", "seq"))
def _sc_lookup(ids, feat, tbl, wfc, *, num_atoms, batch, seq):
    n_rows = batch * seq
    rows_per_w = n_rows // NW
    b_per_w = batch // NW
    half = (seq // 2 + 7) // 8 * 8
    l_starts = (0, seq - half)
    n_fullg = half // LANES
    tailg = half - n_fullg * LANES
    mesh = plsc.VectorSubcoreMesh(
        core_axis_name="c", subcore_axis_name="s", num_cores=NC, num_subcores=NS
    )

    @functools.partial(
        pl.kernel,
        out_type=jax.ShapeDtypeStruct((n_rows, D), jnp.float32),
        mesh=mesh,
        compiler_params=pltpu.CompilerParams(needs_layout_passes=False),
        scratch_types=[
            pltpu.VMEM((num_atoms * D,), jnp.float32),
            pltpu.VMEM((NF, D), jnp.float32),
            pltpu.VMEM((rows_per_w // D, D), jnp.int32),
            pltpu.VMEM((half, NF, b_per_w), jnp.float32),
            pltpu.VMEM((half, D), jnp.float32),
        ],
    )
    def body(ids_hbm, feat_hbm, tbl_hbm, wfc_hbm, out_hbm, tbl_v, wfc_v, ids_v, feat_v, out_v):
        wid = lax.axis_index("s") * NC + lax.axis_index("c")
        base0 = wid * rows_per_w
        pltpu.sync_copy(tbl_hbm, tbl_v)
        pltpu.sync_copy(wfc_hbm, wfc_v)
        pltpu.sync_copy(
            ids_hbm.at[pl.ds(pl.multiple_of(base0 // D, 8), rows_per_w // D)], ids_v
        )
        cols = [lax.iota(jnp.int32, LANES) + LANES * j for j in range(D // LANES)]
        iota = lax.iota(jnp.int32, LANES)
        w = [
            [wfc_v[k, pl.ds(LANES * j, LANES)] for j in range(D // LANES)]
            for k in range(NF)
        ]

        def do_group(bb, l0, lrel0):
            r0 = bb * seq + l0 + lrel0
            rvec = r0 + iota
            idbase = plsc.load_gather(ids_v, [rvec // D, rvec % D]) * D
            lvec = lrel0 + iota
            fk = [
                plsc.load_gather(
                    feat_v,
                    [lvec, jnp.full((LANES,), q, jnp.int32),
                     jnp.full((LANES,), bb, jnp.int32)],
                )
                for q in range(NF)
            ]
            for rr in range(LANES):
                f0 = fk[0][rr]
                f1 = fk[1][rr]
                f2 = fk[2][rr]
                f3 = fk[3][rr]
                for j in range(D // LANES):
                    g16 = plsc.load_gather(tbl_v, [idbase[rr] + cols[j]])
                    acc = (g16 + (f0 * w[0][j] + f1 * w[1][j])) + (
                        f2 * w[2][j] + f3 * w[3][j]
                    )
                    out_v[lrel0 + rr, pl.ds(LANES * j, LANES)] = acc

        for l0 in l_starts:
            pltpu.sync_copy(
                feat_hbm.at[pl.ds(l0, half), :, pl.ds(wid * b_per_w, b_per_w)],
                feat_v,
            )

            def bb_body(bb, carry):
                @plsc.parallel_loop(0, n_fullg, unroll=2)
                def group_body(g):
                    do_group(bb, l0, g * LANES)

                if tailg:
                    do_group(bb, l0, half - LANES)

                pltpu.sync_copy(
                    out_v,
                    out_hbm.at[
                        pl.ds(pl.multiple_of(base0 + bb * seq + l0, 8), half)
                    ],
                )
                return carry

            lax.fori_loop(0, b_per_w, bb_body, 0)

    return body(ids, feat, tbl, wfc)


def kernel(atom_ids, atom_features, table, Wf, bf, Wc, bc):
    B, L = atom_ids.shape
    n_rows = B * L
    ids = atom_ids.reshape(n_rows // D, D).astype(jnp.int32)
    feat = atom_features.transpose(1, 2, 0)
    bfc = jnp.stack([bf, bc])
    tbl, wfc = _prep(table, Wf, bfc, Wc)
    out = _sc_lookup(
        ids, feat, tbl.reshape(-1), wfc,
        num_atoms=table.shape[0], batch=B, seq=L,
    )
    return out.reshape(B, L, D)


# speedup vs baseline: 1.4850x; 1.1688x over previous
import functools

import jax
import jax.numpy as jnp
from jax import lax
from jax.experimental import pallas as pl
from jax.experimental.pallas import tpu as pltpu
from jax.experimental.pallas import tpu_sc as plsc

D = 128
NF = 4
LANES = 16
NC, NS = 2, 16
NW = NC * NS


def _prep_body(table_ref, wf_ref, bfc_ref, wc_ref, tbl_out_ref, wfc_out_ref):
    wc1 = wc_ref[:D, :]
    wc2 = wc_ref[D:, :]
    bias = (
        jnp.dot(bfc_ref[0:1, :], wc2, preferred_element_type=jnp.float32)
        + bfc_ref[1:2, :]
    )
    tbl_out_ref[...] = (
        jnp.dot(table_ref[...], wc1, preferred_element_type=jnp.float32) + bias
    )
    wfc_out_ref[...] = jnp.dot(wf_ref[...], wc2, preferred_element_type=jnp.float32)


def _prep(table, Wf, bfc, Wc):
    num_atoms = table.shape[0]
    return pl.pallas_call(
        _prep_body,
        out_shape=[
            jax.ShapeDtypeStruct((num_atoms, D), jnp.float32),
            jax.ShapeDtypeStruct((NF, D), jnp.float32),
        ],
    )(table, Wf, bfc, Wc)


@functools.partial(jax.jit, static_argnames=("num_atoms", "batch", "seq"))
def _sc_lookup(ids, feat, tbl, wfc, *, num_atoms, batch, seq):
    n_rows = batch * seq
    rows_per_w = n_rows // NW
    b_per_w = batch // NW
    half = -(seq // -(2 * LANES)) * LANES
    l_step = seq - half
    n_fullg = half // LANES
    mesh = plsc.VectorSubcoreMesh(
        core_axis_name="c", subcore_axis_name="s", num_cores=NC, num_subcores=NS
    )

    @functools.partial(
        pl.kernel,
        out_type=jax.ShapeDtypeStruct((n_rows, D), jnp.float32),
        mesh=mesh,
        compiler_params=pltpu.CompilerParams(needs_layout_passes=False),
        scratch_types=[
            pltpu.VMEM((num_atoms * D,), jnp.float32),
            pltpu.VMEM((NF, D), jnp.float32),
            pltpu.VMEM((rows_per_w // D, D), jnp.int32),
            pltpu.VMEM((half, NF, b_per_w), jnp.float32),
            pltpu.VMEM((half, D), jnp.float32),
            pltpu.VMEM((half, D), jnp.float32),
            pltpu.SemaphoreType.DMA,
        ],
    )
    def body(ids_hbm, feat_hbm, tbl_hbm, wfc_hbm, out_hbm, tbl_v, wfc_v, ids_v, feat_v, out_a, out_b, sem):
        wid = lax.axis_index("s") * NC + lax.axis_index("c")
        base0 = wid * rows_per_w
        pltpu.sync_copy(tbl_hbm, tbl_v)
        pltpu.sync_copy(wfc_hbm, wfc_v)
        pltpu.sync_copy(
            ids_hbm.at[pl.ds(pl.multiple_of(base0 // D, 8), rows_per_w // D)], ids_v
        )
        cols = [lax.iota(jnp.int32, LANES) + LANES * j for j in range(D // LANES)]
        iota = lax.iota(jnp.int32, LANES)
        w = [
            [wfc_v[k, pl.ds(LANES * j, LANES)] for j in range(D // LANES)]
            for k in range(NF)
        ]

        tbl_slices = [
            tbl_v.at[pl.ds(LANES * j, (num_atoms - 1) * D + LANES)]
            for j in range(D // LANES)
        ]

        def do_group(bb, l0, lrel0, out_v):
            r0 = bb * seq + l0 + lrel0
            rvec = r0 + iota
            idbase = plsc.load_gather(ids_v, [rvec // D, rvec % D]) * D
            lvec = lrel0 + iota
            fk = [
                plsc.load_gather(
                    feat_v,
                    [lvec, jnp.full((LANES,), q, jnp.int32),
                     jnp.full((LANES,), bb, jnp.int32)],
                )
                for q in range(NF)
            ]
            for rr in range(LANES):
                f0 = fk[0][rr]
                f1 = fk[1][rr]
                f2 = fk[2][rr]
                f3 = fk[3][rr]
                idx = idbase[rr] + iota
                for j in range(D // LANES):
                    g16 = plsc.load_gather(tbl_slices[j], [idx])
                    acc = (g16 + (f0 * w[0][j] + f1 * w[1][j])) + (
                        f2 * w[2][j] + f3 * w[3][j]
                    )
                    out_v[lrel0 + rr, pl.ds(LANES * j, LANES)] = acc

        def compute(bb, l0, out_v):
            @plsc.parallel_loop(0, n_fullg, unroll=2)
            def group_body(g):
                do_group(bb, l0, g * LANES, out_v)

        def out_slice(bb, l0):
            return out_hbm.at[
                pl.ds(pl.multiple_of(base0 + bb * seq + l0, 8), half)
            ]

        def issue(bb, l0, out_v):
            pltpu.async_copy(out_v, out_slice(bb, l0), sem)

        def wait_one(l0):
            pltpu.make_async_copy(out_a, out_slice(0, l0), sem).wait()

        def l_body(li, carry):
            l0 = li * l_step
            pltpu.sync_copy(
                feat_hbm.at[
                    pl.ds(pl.multiple_of(l0, 8), half),
                    slice(None),
                    pl.ds(wid * b_per_w, b_per_w),
                ],
                feat_v,
            )
            compute(0, l0, out_a)
            issue(0, l0, out_a)
            compute(1, l0, out_b)
            issue(1, l0, out_b)

            def pair_body(p, c):
                bb = 2 * p
                wait_one(l0)
                compute(bb, l0, out_a)
                issue(bb, l0, out_a)
                wait_one(l0)
                compute(bb + 1, l0, out_b)
                issue(bb + 1, l0, out_b)
                return c

            lax.fori_loop(1, b_per_w // 2, pair_body, 0)
            wait_one(l0)
            wait_one(l0)
            return carry

        lax.fori_loop(0, 2, l_body, 0)

    return body(ids, feat, tbl, wfc)


def kernel(atom_ids, atom_features, table, Wf, bf, Wc, bc):
    B, L = atom_ids.shape
    n_rows = B * L
    ids = atom_ids.reshape(n_rows // D, D).astype(jnp.int32)
    feat = atom_features.transpose(1, 2, 0)
    bfc = jnp.stack([bf, bc])
    tbl, wfc = _prep(table, Wf, bfc, Wc)
    out = _sc_lookup(
        ids, feat, tbl.reshape(-1), wfc,
        num_atoms=table.shape[0], batch=B, seq=L,
    )
    return out.reshape(B, L, D)


# speedup vs baseline: 2.8436x; 1.9148x over previous
import functools

import jax
import jax.numpy as jnp
from jax import lax
from jax.experimental import pallas as pl
from jax.experimental.pallas import tpu as pltpu
from jax.experimental.pallas import tpu_sc as plsc

D = 128
NF = 4
LANES = 16
NC, NS = 2, 16
NW = NC * NS


def _prep_body(table_ref, wf_ref, bfc_ref, wc_ref, tbl_out_ref, wfc_out_ref):
    wc1 = wc_ref[:D, :]
    wc2 = wc_ref[D:, :]
    bias = (
        jnp.dot(bfc_ref[0:1, :], wc2, preferred_element_type=jnp.float32)
        + bfc_ref[1:2, :]
    )
    tbl_out_ref[...] = (
        jnp.dot(table_ref[...], wc1, preferred_element_type=jnp.float32) + bias
    )
    wfc_out_ref[...] = jnp.dot(wf_ref[...], wc2, preferred_element_type=jnp.float32)


def _prep(table, Wf, bfc, Wc):
    num_atoms = table.shape[0]
    return pl.pallas_call(
        _prep_body,
        out_shape=[
            jax.ShapeDtypeStruct((num_atoms, D), jnp.float32),
            jax.ShapeDtypeStruct((NF, D), jnp.float32),
        ],
    )(table, Wf, bfc, Wc)


@functools.partial(jax.jit, static_argnames=("num_atoms", "batch", "seq"))
def _sc_lookup(ids, feat, tbl, wfc, *, num_atoms, batch, seq):
    n_rows = batch * seq
    rows_per_w = n_rows // NW
    b_per_w = batch // NW
    half = -(seq // -(2 * LANES)) * LANES
    l_step = seq - half
    n_fullg = half // LANES
    mesh = plsc.VectorSubcoreMesh(
        core_axis_name="c", subcore_axis_name="s", num_cores=NC, num_subcores=NS
    )

    @functools.partial(
        pl.kernel,
        out_type=jax.ShapeDtypeStruct((n_rows, D), jnp.float32),
        mesh=mesh,
        compiler_params=pltpu.CompilerParams(needs_layout_passes=False),
        scratch_types=[
            pltpu.VMEM((num_atoms * D,), jnp.float32),
            pltpu.VMEM((NF, D), jnp.float32),
            pltpu.VMEM((rows_per_w // D, D), jnp.int32),
            pltpu.VMEM((half, NF, b_per_w), jnp.float32),
            pltpu.VMEM((half, D), jnp.float32),
            pltpu.VMEM((half, D), jnp.float32),
            pltpu.SemaphoreType.DMA,
        ],
    )
    def body(ids_hbm, feat_hbm, tbl_hbm, wfc_hbm, out_hbm, tbl_v, wfc_v, ids_v, feat_v, out_a, out_b, sem):
        wid = lax.axis_index("s") * NC + lax.axis_index("c")
        base0 = wid * rows_per_w
        pltpu.sync_copy(tbl_hbm, tbl_v)
        pltpu.sync_copy(wfc_hbm, wfc_v)
        pltpu.sync_copy(
            ids_hbm.at[pl.ds(pl.multiple_of(base0 // D, 8), rows_per_w // D)], ids_v
        )
        cols = [lax.iota(jnp.int32, LANES) + LANES * j for j in range(D // LANES)]
        iota = lax.iota(jnp.int32, LANES)
        w = [
            [wfc_v[k, pl.ds(LANES * j, LANES)] for j in range(D // LANES)]
            for k in range(NF)
        ]
        wp = [
            [
                plsc.pack(w[k][2 * jj], w[k][2 * jj + 1],
                          format=plsc.PackFormat.INTERLEAVED)
                for jj in range(D // (2 * LANES))
            ]
            for k in range(NF)
        ]

        tbl_slices = [
            tbl_v.at[pl.ds(LANES * j, (num_atoms - 1) * D + LANES)]
            for j in range(D // LANES)
        ]

        def do_group(bb, l0, lrel0, out_v):
            r0 = bb * seq + l0 + lrel0
            rvec = r0 + iota
            idbase = plsc.load_gather(ids_v, [rvec // D, rvec % D]) * D
            lvec = lrel0 + iota
            fk = [
                plsc.load_gather(
                    feat_v,
                    [lvec, jnp.full((LANES,), q, jnp.int32),
                     jnp.full((LANES,), bb, jnp.int32)],
                )
                for q in range(NF)
            ]
            for rr in range(LANES):
                fp = []
                for q in range(NF):
                    fv = jnp.full((LANES,), fk[q][rr], jnp.float32)
                    fp.append(
                        plsc.pack(fv, fv, format=plsc.PackFormat.INTERLEAVED)
                    )
                idx = idbase[rr] + iota
                for jj in range(D // (2 * LANES)):
                    s = (fp[0] * wp[0][jj] + fp[1] * wp[1][jj]) + (
                        fp[2] * wp[2][jj] + fp[3] * wp[3][jj]
                    )
                    s0, s1 = plsc.unpack(s, format=plsc.PackFormat.INTERLEAVED)
                    g0 = plsc.load_gather(tbl_slices[2 * jj], [idx])
                    g1 = plsc.load_gather(tbl_slices[2 * jj + 1], [idx])
                    out_v[lrel0 + rr, pl.ds(LANES * 2 * jj, LANES)] = g0 + s0
                    out_v[lrel0 + rr, pl.ds(LANES * (2 * jj + 1), LANES)] = g1 + s1

        def compute(bb, l0, out_v):
            @plsc.parallel_loop(0, n_fullg, unroll=2)
            def group_body(g):
                do_group(bb, l0, g * LANES, out_v)

        def out_slice(bb, l0):
            return out_hbm.at[
                pl.ds(pl.multiple_of(base0 + bb * seq + l0, 8), half)
            ]

        def issue(bb, l0, out_v):
            pltpu.async_copy(out_v, out_slice(bb, l0), sem)

        def wait_one(l0):
            pltpu.make_async_copy(out_a, out_slice(0, l0), sem).wait()

        def l_body(li, carry):
            l0 = li * l_step
            pltpu.sync_copy(
                feat_hbm.at[
                    pl.ds(pl.multiple_of(l0, 8), half),
                    slice(None),
                    pl.ds(wid * b_per_w, b_per_w),
                ],
                feat_v,
            )
            compute(0, l0, out_a)
            issue(0, l0, out_a)
            compute(1, l0, out_b)
            issue(1, l0, out_b)

            def pair_body(p, c):
                bb = 2 * p
                wait_one(l0)
                compute(bb, l0, out_a)
                issue(bb, l0, out_a)
                wait_one(l0)
                compute(bb + 1, l0, out_b)
                issue(bb + 1, l0, out_b)
                return c

            lax.fori_loop(1, b_per_w // 2, pair_body, 0)
            wait_one(l0)
            wait_one(l0)
            return carry

        lax.fori_loop(0, 2, l_body, 0)

    return body(ids, feat, tbl, wfc)


def kernel(atom_ids, atom_features, table, Wf, bf, Wc, bc):
    B, L = atom_ids.shape
    n_rows = B * L
    ids = atom_ids.reshape(n_rows // D, D).astype(jnp.int32)
    feat = atom_features.transpose(1, 2, 0)
    bfc = jnp.stack([bf, bc])
    tbl, wfc = _prep(table, Wf, bfc, Wc)
    out = _sc_lookup(
        ids, feat, tbl.reshape(-1), wfc,
        num_atoms=table.shape[0], batch=B, seq=L,
    )
    return out.reshape(B, L, D)
